# baseline (speedup 1.0000x reference)
import jax
import jax.numpy as jnp
from jax.experimental import pallas as pl
from jax.experimental.pallas import tpu as pltpu

_K = 10
_N = 10000
_D = 128
_NS = 10400
_RT1 = 400
_T1 = _N // _RT1
_RT2 = 400
_T2 = -(-_N // _RT2)


def _hop0_body(temp_ref, x_ref, a_ref, ab_ref, h1_ref, acc1_ref):
    t = pl.program_id(0)
    a_bf = a_ref[...].astype(jnp.bfloat16)
    ab_ref[...] = a_bf
    h_new = jnp.dot(a_bf, x_ref[...],
                    preferred_element_type=jnp.float32)
    h1_ref[...] = h_new.astype(jnp.bfloat16)
    x_rows = x_ref[pl.ds(t * _RT1, _RT1), :].astype(jnp.float32)
    acc1_ref[...] = temp_ref[0] * x_rows + temp_ref[1] * h_new


_HRT = _RT2 // 2


def _hops_body(temp_ref, h1_ref, acc1_ref, ab0_ref, ab1_ref, out_ref,
               acc_ref, h_ref):
    k = pl.program_id(0)
    t = pl.program_id(1)

    @pl.when((k == 0) & (t == 0))
    def _init():
        for tt in range(_T2):
            n_rows = min(_RT2, _N - tt * _RT2)
            h_ref[0, tt, pl.ds(0, n_rows), :] = (
                h1_ref[pl.ds(tt * _RT2, n_rows), :])
            acc_ref[tt, pl.ds(0, n_rows), :] = (
                acc1_ref[pl.ds(tt * _RT2, n_rows), :])

    rd = jax.lax.rem(k, 2)
    wr = 1 - rd

    h_old = h_ref[rd].reshape(_T2 * _RT2, _D)[:_N]
    tk = temp_ref[k + 2]
    for half, a_ref in ((0, ab0_ref), (1, ab1_ref)):
        h_new = jnp.dot(a_ref[...], h_old,
                        preferred_element_type=jnp.float32)
        r0 = half * _HRT
        h_ref[wr, t, pl.ds(r0, _HRT), :] = h_new.astype(jnp.bfloat16)
        acc_ref[t, pl.ds(r0, _HRT), :] = (
            acc_ref[t, pl.ds(r0, _HRT), :] + tk * h_new
        )

    @pl.when(k == _K - 2)
    def _emit():
        out_ref[...] = acc_ref[t]


def kernel(x, A_hat, temp):
    x_b = x.astype(jnp.bfloat16)

    hop0 = pltpu.PrefetchScalarGridSpec(
        num_scalar_prefetch=1,
        grid=(_T1,),
        in_specs=[
            pl.BlockSpec((_N, _D), lambda t, *_: (0, 0)),
            pl.BlockSpec((_RT1, _N), lambda t, *_: (t, 0)),
        ],
        out_specs=[
            pl.BlockSpec((_RT1, _N), lambda t, *_: (t, 0)),
            pl.BlockSpec((_RT1, _D), lambda t, *_: (t, 0)),
            pl.BlockSpec((_RT1, _D), lambda t, *_: (t, 0)),
        ],
    )
    a_b, h1, acc1 = pl.pallas_call(
        _hop0_body,
        grid_spec=hop0,
        out_shape=[
            jax.ShapeDtypeStruct((_N, _N), jnp.bfloat16),
            jax.ShapeDtypeStruct((_N, _D), jnp.bfloat16),
            jax.ShapeDtypeStruct((_N, _D), jnp.float32),
        ],
        compiler_params=pltpu.CompilerParams(
            dimension_semantics=("arbitrary",),
        ),
    )(temp, x_b, A_hat)

    hops = pltpu.PrefetchScalarGridSpec(
        num_scalar_prefetch=1,
        grid=(_K - 1, _T2),
        in_specs=[
            pl.BlockSpec((_N, _D), lambda k, t, *_: (0, 0)),
            pl.BlockSpec((_N, _D), lambda k, t, *_: (0, 0)),
            pl.BlockSpec((_HRT, _N),
                         lambda k, t, *_: (2 * t, 0)),
            pl.BlockSpec((_HRT, _N),
                         lambda k, t, *_: (jnp.minimum(2 * t + 1, _N // _HRT - 1),
                                           0)),
        ],
        out_specs=pl.BlockSpec(
            (_RT2, _D), lambda k, t, *_: (jnp.where(k == _K - 2, t, 0), 0)),
        scratch_shapes=[
            pltpu.VMEM((_T2, _RT2, _D), jnp.float32),
            pltpu.VMEM((2, _T2, _RT2, _D), jnp.bfloat16),
        ],
    )
    out = pl.pallas_call(
        _hops_body,
        grid_spec=hops,
        out_shape=jax.ShapeDtypeStruct((_N, _D), jnp.float32),
        compiler_params=pltpu.CompilerParams(
            dimension_semantics=("arbitrary", "arbitrary"),
        ),
    )(temp, h1, acc1, a_b, a_b)

    return out


# speedup vs baseline: 1.0734x; 1.0734x over previous
import jax
import jax.numpy as jnp
from jax.experimental import pallas as pl
from jax.experimental.pallas import tpu as pltpu

_K = 10
_N = 10000
_D = 128
_RT1 = 400
_T1 = _N // _RT1
_RT2 = 800
_T2 = -(-_N // _RT2)


def _hop0_body(temp_ref, x_ref, a_ref, ab_ref, h1_ref, acc1_ref):
    t = pl.program_id(0)
    a_bf = a_ref[...].astype(jnp.bfloat16)
    ab_ref[...] = a_bf
    h_new = jnp.dot(a_bf, x_ref[...],
                    preferred_element_type=jnp.float32)
    h1_ref[...] = h_new.astype(jnp.bfloat16)
    x_rows = x_ref[pl.ds(t * _RT1, _RT1), :].astype(jnp.float32)
    acc1_ref[...] = temp_ref[0] * x_rows + temp_ref[1] * h_new


_HRT = _RT2 // 2


def _hops_body(temp_ref, h1_ref, acc1_ref, ab0_ref, ab1_ref, out_ref,
               acc_ref, h_ref):
    k = pl.program_id(0)
    t = pl.program_id(1)

    @pl.when((k == 0) & (t == 0))
    def _init():
        for tt in range(_T2):
            n_rows = min(_RT2, _N - tt * _RT2)
            h_ref[0, tt, pl.ds(0, n_rows), :] = (
                h1_ref[pl.ds(tt * _RT2, n_rows), :])
            acc_ref[tt, pl.ds(0, n_rows), :] = (
                acc1_ref[pl.ds(tt * _RT2, n_rows), :])

    rd = jax.lax.rem(k, 2)
    wr = 1 - rd

    h_old = h_ref[rd].reshape(_T2 * _RT2, _D)[:_N]
    tk = temp_ref[k + 2]
    for half, a_ref in ((0, ab0_ref), (1, ab1_ref)):
        h_new = jnp.dot(a_ref[...], h_old,
                        preferred_element_type=jnp.float32)
        r0 = half * _HRT
        h_ref[wr, t, pl.ds(r0, _HRT), :] = h_new.astype(jnp.bfloat16)
        acc_ref[t, pl.ds(r0, _HRT), :] = (
            acc_ref[t, pl.ds(r0, _HRT), :] + tk * h_new
        )

    @pl.when(k == _K - 2)
    def _emit():
        out_ref[...] = acc_ref[t]


def kernel(x, A_hat, temp):
    x_b = x.astype(jnp.bfloat16)

    hop0 = pltpu.PrefetchScalarGridSpec(
        num_scalar_prefetch=1,
        grid=(_T1,),
        in_specs=[
            pl.BlockSpec((_N, _D), lambda t, *_: (0, 0)),
            pl.BlockSpec((_RT1, _N), lambda t, *_: (t, 0)),
        ],
        out_specs=[
            pl.BlockSpec((_RT1, _N), lambda t, *_: (t, 0)),
            pl.BlockSpec((_RT1, _D), lambda t, *_: (t, 0)),
            pl.BlockSpec((_RT1, _D), lambda t, *_: (t, 0)),
        ],
    )
    a_b, h1, acc1 = pl.pallas_call(
        _hop0_body,
        grid_spec=hop0,
        out_shape=[
            jax.ShapeDtypeStruct((_N, _N), jnp.bfloat16),
            jax.ShapeDtypeStruct((_N, _D), jnp.bfloat16),
            jax.ShapeDtypeStruct((_N, _D), jnp.float32),
        ],
        compiler_params=pltpu.CompilerParams(
            dimension_semantics=("arbitrary",),
        ),
    )(temp, x_b, A_hat)

    hops = pltpu.PrefetchScalarGridSpec(
        num_scalar_prefetch=1,
        grid=(_K - 1, _T2),
        in_specs=[
            pl.BlockSpec((_N, _D), lambda k, t, *_: (0, 0)),
            pl.BlockSpec((_N, _D), lambda k, t, *_: (0, 0)),
            pl.BlockSpec((_HRT, _N),
                         lambda k, t, *_: (2 * t, 0)),
            pl.BlockSpec((_HRT, _N),
                         lambda k, t, *_: (jnp.minimum(2 * t + 1, _N // _HRT - 1),
                                           0)),
        ],
        out_specs=pl.BlockSpec(
            (_RT2, _D), lambda k, t, *_: (jnp.where(k == _K - 2, t, 0), 0)),
        scratch_shapes=[
            pltpu.VMEM((_T2, _RT2, _D), jnp.float32),
            pltpu.VMEM((2, _T2, _RT2, _D), jnp.bfloat16),
        ],
    )
    out = pl.pallas_call(
        _hops_body,
        grid_spec=hops,
        out_shape=jax.ShapeDtypeStruct((_N, _D), jnp.float32),
        compiler_params=pltpu.CompilerParams(
            dimension_semantics=("arbitrary", "arbitrary"),
        ),
    )(temp, h1, acc1, a_b, a_b)

    return out
